# initial kernel scaffold (unmeasured)
import jax
import jax.numpy as jnp
from jax import lax
from jax.experimental import pallas as pl
from jax.experimental.pallas import tpu as pltpu


def kernel(
    x,
):
    def body(*refs):
        pass

    out_shape = jax.ShapeDtypeStruct(..., jnp.float32)
    return pl.pallas_call(body, out_shape=out_shape)(...)


# baseline (device time: 32041 ns/iter reference)
import jax
import jax.numpy as jnp
from jax import lax
from jax.experimental import pallas as pl
from jax.experimental.pallas import tpu as pltpu

N_DEV = 4


def kernel(x):
    m, n = x.shape
    chunk = m // N_DEV

    def body(x_ref, out_ref, comm_ref, send_sems, recv_sems):
        my = lax.axis_index("i")
        left = lax.rem(my + N_DEV - 1, N_DEV)
        right = lax.rem(my + 1, N_DEV)

        barrier_sem = pltpu.get_barrier_semaphore()
        for nbr in (left, right):
            pl.semaphore_signal(
                barrier_sem, inc=1,
                device_id=(nbr,), device_id_type=pl.DeviceIdType.MESH,
            )
        pl.semaphore_wait(barrier_sem, 2)

        out_ref[...] = x_ref[...]

        for s in range(N_DEV - 1):
            send_idx = lax.rem(my + (2 * N_DEV - s), N_DEV)
            rdma = pltpu.make_async_remote_copy(
                src_ref=out_ref.at[pl.ds(send_idx * chunk, chunk)],
                dst_ref=comm_ref.at[s],
                send_sem=send_sems.at[s],
                recv_sem=recv_sems.at[s],
                device_id=(right,),
                device_id_type=pl.DeviceIdType.MESH,
            )
            rdma.start()
            rdma.wait()
            recv_idx = lax.rem(my + (2 * N_DEV - s - 1), N_DEV)
            sl = pl.ds(recv_idx * chunk, chunk)
            out_ref[sl, :] = out_ref[sl, :] + comm_ref[s]

        for t in range(N_DEV - 1):
            send_idx = lax.rem(my + (2 * N_DEV + 1 - t), N_DEV)
            sl = pl.ds(send_idx * chunk, chunk)
            k = (N_DEV - 1) + t
            rdma = pltpu.make_async_remote_copy(
                src_ref=out_ref.at[sl],
                dst_ref=out_ref.at[sl],
                send_sem=send_sems.at[k],
                recv_sem=recv_sems.at[k],
                device_id=(right,),
                device_id_type=pl.DeviceIdType.MESH,
            )
            rdma.start()
            rdma.wait()

    return pl.pallas_call(
        body,
        out_shape=jax.ShapeDtypeStruct((m, n), x.dtype),
        in_specs=[pl.BlockSpec(memory_space=pltpu.VMEM)],
        out_specs=pl.BlockSpec(memory_space=pltpu.VMEM),
        scratch_shapes=[
            pltpu.VMEM((N_DEV - 1, chunk, n), x.dtype),
            pltpu.SemaphoreType.DMA((2 * (N_DEV - 1),)),
            pltpu.SemaphoreType.DMA((2 * (N_DEV - 1),)),
        ],
        compiler_params=pltpu.CompilerParams(collective_id=0),
    )(x)


# device time: 20372 ns/iter; 1.5728x vs baseline; 1.5728x over previous
import jax
import jax.numpy as jnp
from jax import lax
from jax.experimental import pallas as pl
from jax.experimental.pallas import tpu as pltpu

N_DEV = 4


def kernel(x):
    m, n = x.shape
    half = m // 2
    quar = m // 4
    nh = n // 2

    def body(x_ref, out_ref, buf1, buf2, send_sems, recv_sems):
        p = lax.axis_index("i")
        p1 = jnp.bitwise_xor(p, 1)
        p2 = 3 - p

        barrier_sem = pltpu.get_barrier_semaphore()
        for nbr in (p1, p2):
            pl.semaphore_signal(
                barrier_sem, inc=1,
                device_id=(nbr,), device_id_type=pl.DeviceIdType.MESH,
            )
        pl.semaphore_wait(barrier_sem, 2)

        out_ref[...] = x_ref[...]

        keep_a = jnp.where((p == 0) | (p == 3), 0, half)
        keep_b = jnp.where(p < 2, 0, half)
        own_a = keep_a + jnp.where(p >= 2, quar, 0)
        own_b = keep_b + jnp.where((p == 1) | (p == 3), quar, 0)
        part_a = 2 * keep_a + quar - own_a
        part_b = 2 * keep_b + quar - own_b

        ca = pl.ds(0, nh)
        cb = pl.ds(nh, nh)

        def copy(src_r, src_c, dst_ref, dst_r, dst_c, sem, dev):
            return pltpu.make_async_remote_copy(
                src_ref=out_ref.at[src_r, src_c],
                dst_ref=dst_ref.at[dst_r, dst_c] if dst_r is not None else dst_ref,
                send_sem=send_sems.at[sem],
                recv_sem=recv_sems.at[sem],
                device_id=(dev,),
                device_id_type=pl.DeviceIdType.MESH,
            )

        ra = copy(pl.ds(half - keep_a, half), ca, buf1.at[0], None, None, 0, p1)
        rb = copy(pl.ds(half - keep_b, half), cb, buf1.at[1], None, None, 1, p2)
        ra.start()
        rb.start()
        ra.wait()
        rb.wait()
        sa = pl.ds(keep_a, half)
        sb = pl.ds(keep_b, half)
        out_ref[sa, ca] = out_ref[sa, ca] + buf1[0]
        out_ref[sb, cb] = out_ref[sb, cb] + buf1[1]

        ra = copy(pl.ds(part_a, quar), ca, buf2.at[0], None, None, 2, p2)
        rb = copy(pl.ds(part_b, quar), cb, buf2.at[1], None, None, 3, p1)
        ra.start()
        rb.start()
        ra.wait()
        rb.wait()
        qa = pl.ds(own_a, quar)
        qb = pl.ds(own_b, quar)
        out_ref[qa, ca] = out_ref[qa, ca] + buf2[0]
        out_ref[qb, cb] = out_ref[qb, cb] + buf2[1]

        ra = copy(qa, ca, out_ref, qa, ca, 4, p2)
        rb = copy(qb, cb, out_ref, qb, cb, 5, p1)
        ra.start()
        rb.start()
        ra.wait()
        rb.wait()

        ra = copy(sa, ca, out_ref, sa, ca, 6, p1)
        rb = copy(sb, cb, out_ref, sb, cb, 7, p2)
        ra.start()
        rb.start()
        ra.wait()
        rb.wait()

    return pl.pallas_call(
        body,
        out_shape=jax.ShapeDtypeStruct((m, n), x.dtype),
        in_specs=[pl.BlockSpec(memory_space=pltpu.VMEM)],
        out_specs=pl.BlockSpec(memory_space=pltpu.VMEM),
        scratch_shapes=[
            pltpu.VMEM((2, half, nh), x.dtype),
            pltpu.VMEM((2, quar, nh), x.dtype),
            pltpu.SemaphoreType.DMA((8,)),
            pltpu.SemaphoreType.DMA((8,)),
        ],
        compiler_params=pltpu.CompilerParams(collective_id=0),
    )(x)


# device time: 20265 ns/iter; 1.5811x vs baseline; 1.0053x over previous
import jax
import jax.numpy as jnp
from jax import lax
from jax.experimental import pallas as pl
from jax.experimental.pallas import tpu as pltpu

N_DEV = 4


def kernel(x):
    m, n = x.shape
    half = m // 2
    quar = m // 4
    nh = n // 2

    def body(x_ref, out_ref, buf1, buf2, send_sems, recv_sems):
        p = lax.axis_index("i")
        p1 = jnp.bitwise_xor(p, 1)
        p2 = 3 - p

        barrier_sem = pltpu.get_barrier_semaphore()
        for nbr in (p1, p2):
            pl.semaphore_signal(
                barrier_sem, inc=1,
                device_id=(nbr,), device_id_type=pl.DeviceIdType.MESH,
            )
        pl.semaphore_wait(barrier_sem, 2)

        keep_a = jnp.where((p == 0) | (p == 3), 0, half)
        keep_b = jnp.where(p < 2, 0, half)
        own_a = keep_a + jnp.where(p >= 2, quar, 0)
        own_b = keep_b + jnp.where((p == 1) | (p == 3), quar, 0)
        part_a = 2 * keep_a + quar - own_a
        part_b = 2 * keep_b + quar - own_b

        ca = pl.ds(0, nh)
        cb = pl.ds(nh, nh)

        def copy(src_ref, src_r, src_c, dst_ref, dst_r, dst_c, sem, dev):
            return pltpu.make_async_remote_copy(
                src_ref=src_ref.at[src_r, src_c] if src_r is not None else src_ref,
                dst_ref=dst_ref.at[dst_r, dst_c] if dst_r is not None else dst_ref,
                send_sem=send_sems.at[sem],
                recv_sem=recv_sems.at[sem],
                device_id=(dev,),
                device_id_type=pl.DeviceIdType.MESH,
            )

        sa, sb = pl.ds(keep_a, half), pl.ds(keep_b, half)
        qa, qb = pl.ds(own_a, quar), pl.ds(own_b, quar)

        r_a1 = copy(x_ref, pl.ds(half - keep_a, half), ca, buf1.at[0], None, None, 0, p1)
        r_b1 = copy(x_ref, pl.ds(half - keep_b, half), cb, buf1.at[1], None, None, 1, p2)
        r_a1.start()
        r_b1.start()

        r_a1.wait_recv()
        out_ref[sa, ca] = x_ref[sa, ca] + buf1[0]
        r_a2 = copy(out_ref, pl.ds(part_a, quar), ca, buf2.at[0], None, None, 2, p2)
        r_a2.start()

        r_b1.wait_recv()
        out_ref[sb, cb] = x_ref[sb, cb] + buf1[1]
        r_b2 = copy(out_ref, pl.ds(part_b, quar), cb, buf2.at[1], None, None, 3, p1)
        r_b2.start()

        r_a2.wait_recv()
        out_ref[qa, ca] = out_ref[qa, ca] + buf2[0]
        r_a3 = copy(out_ref, qa, ca, out_ref, qa, ca, 4, p2)
        r_a3.start()

        r_b2.wait_recv()
        out_ref[qb, cb] = out_ref[qb, cb] + buf2[1]
        r_b3 = copy(out_ref, qb, cb, out_ref, qb, cb, 5, p1)
        r_b3.start()

        r_a3.wait_recv()
        r_a4 = copy(out_ref, sa, ca, out_ref, sa, ca, 6, p1)
        r_a4.start()

        r_b3.wait_recv()
        r_b4 = copy(out_ref, sb, cb, out_ref, sb, cb, 7, p2)
        r_b4.start()

        r_a4.wait_recv()
        r_b4.wait_recv()

        for r in (r_a1, r_b1, r_a2, r_b2, r_a3, r_b3, r_a4, r_b4):
            r.wait_send()

    return pl.pallas_call(
        body,
        out_shape=jax.ShapeDtypeStruct((m, n), x.dtype),
        in_specs=[pl.BlockSpec(memory_space=pltpu.VMEM)],
        out_specs=pl.BlockSpec(memory_space=pltpu.VMEM),
        scratch_shapes=[
            pltpu.VMEM((2, half, nh), x.dtype),
            pltpu.VMEM((2, quar, nh), x.dtype),
            pltpu.SemaphoreType.DMA((8,)),
            pltpu.SemaphoreType.DMA((8,)),
        ],
        compiler_params=pltpu.CompilerParams(collective_id=0),
    )(x)


# device time: 17773 ns/iter; 1.8028x vs baseline; 1.1402x over previous
import jax
import jax.numpy as jnp
from jax import lax
from jax.experimental import pallas as pl
from jax.experimental.pallas import tpu as pltpu

N_DEV = 4
C = 2


def kernel(x):
    m, n = x.shape
    half = m // 2
    quar = m // 4
    nh = n // 2
    w = nh // C

    def body(x_ref, out_ref, buf1, buf2, send_sems, recv_sems):
        p = lax.axis_index("i")
        p1 = jnp.bitwise_xor(p, 1)
        p2 = 3 - p

        barrier_sem = pltpu.get_barrier_semaphore()
        for nbr in (p1, p2):
            pl.semaphore_signal(
                barrier_sem, inc=1,
                device_id=(nbr,), device_id_type=pl.DeviceIdType.MESH,
            )
        pl.semaphore_wait(barrier_sem, 2)

        keep_a = jnp.where((p == 0) | (p == 3), 0, half)
        keep_b = jnp.where(p < 2, 0, half)
        own_a = keep_a + jnp.where(p >= 2, quar, 0)
        own_b = keep_b + jnp.where((p == 1) | (p == 3), quar, 0)
        keeps = [keep_a, keep_b]
        owns = [own_a, own_b]
        parts = [2 * keep_a + quar - own_a, 2 * keep_b + quar - own_b]
        orders = [(p1, p2, p2, p1), (p2, p1, p1, p2)]

        def col(d, c):
            return pl.ds(d * nh + c * w, w)

        def sem(phase, d, c):
            return (phase * 2 + d) * C + c

        def copy(src_ref, src_r, src_c, dst_ref, dst_r, dst_c, k, dev):
            return pltpu.make_async_remote_copy(
                src_ref=src_ref.at[src_r, src_c] if src_r is not None else src_ref,
                dst_ref=dst_ref.at[dst_r, dst_c] if dst_r is not None else dst_ref,
                send_sem=send_sems.at[k],
                recv_sem=recv_sems.at[k],
                device_id=(dev,),
                device_id_type=pl.DeviceIdType.MESH,
            )

        chains = [(d, c) for c in range(C) for d in range(2)]
        rd = {}

        for d, c in chains:
            r = copy(x_ref, pl.ds(half - keeps[d], half), col(d, c),
                     buf1.at[d * C + c], None, None, sem(0, d, c), orders[d][0])
            rd[(0, d, c)] = r
            r.start()

        for d, c in chains:
            rd[(0, d, c)].wait_recv()
            s = pl.ds(keeps[d], half)
            out_ref[s, col(d, c)] = x_ref[s, col(d, c)] + buf1[d * C + c]
            r = copy(out_ref, pl.ds(parts[d], quar), col(d, c),
                     buf2.at[d * C + c], None, None, sem(1, d, c), orders[d][1])
            rd[(1, d, c)] = r
            r.start()

        for d, c in chains:
            rd[(1, d, c)].wait_recv()
            q = pl.ds(owns[d], quar)
            out_ref[q, col(d, c)] = out_ref[q, col(d, c)] + buf2[d * C + c]
            r = copy(out_ref, q, col(d, c), out_ref, q, col(d, c),
                     sem(2, d, c), orders[d][2])
            rd[(2, d, c)] = r
            r.start()

        for d, c in chains:
            rd[(2, d, c)].wait_recv()
            s = pl.ds(keeps[d], half)
            r = copy(out_ref, s, col(d, c), out_ref, s, col(d, c),
                     sem(3, d, c), orders[d][3])
            rd[(3, d, c)] = r
            r.start()

        for d, c in chains:
            rd[(3, d, c)].wait_recv()

        for r in rd.values():
            r.wait_send()

    return pl.pallas_call(
        body,
        out_shape=jax.ShapeDtypeStruct((m, n), x.dtype),
        in_specs=[pl.BlockSpec(memory_space=pltpu.VMEM)],
        out_specs=pl.BlockSpec(memory_space=pltpu.VMEM),
        scratch_shapes=[
            pltpu.VMEM((2 * C, half, w), x.dtype),
            pltpu.VMEM((2 * C, quar, w), x.dtype),
            pltpu.SemaphoreType.DMA((8 * C,)),
            pltpu.SemaphoreType.DMA((8 * C,)),
        ],
        compiler_params=pltpu.CompilerParams(collective_id=0),
    )(x)
